# baseline (speedup 1.0000x reference)
import functools

import jax
import jax.numpy as jnp
from jax import lax
from jax.experimental import pallas as pl
from jax.experimental.pallas import tpu as pltpu
from jax.experimental.pallas import tpu_sc as plsc

_LOG2E = 1.4426950408889634
_LN2 = 0.6931471805599453


def _tc_transpose(x_t):
    D, V = x_t.shape
    blk = 16384

    def body(x_ref, o_ref):
        eye = jnp.eye(D, dtype=jnp.float32)
        o_ref[...] = lax.dot_general(
            x_ref[...], eye, (((0,), (0,)), ((), ())),
            preferred_element_type=jnp.float32,
        )

    return pl.pallas_call(
        body,
        grid=(pl.cdiv(V, blk),),
        in_specs=[pl.BlockSpec((D, blk), lambda i: (0, i))],
        out_specs=pl.BlockSpec((blk, D), lambda i: (i, 0)),
        out_shape=jax.ShapeDtypeStruct((V, D), jnp.float32),
    )(x_t)


def _sc_gather(table, idx):
    V, D = table.shape
    B = idx.shape[0]
    info = plsc.get_sparse_core_info()
    nc, ns = info.num_cores, info.num_subcores
    b_per_w = B // (nc * ns)
    mesh = plsc.VectorSubcoreMesh(core_axis_name="c", subcore_axis_name="s")

    @functools.partial(
        pl.kernel,
        mesh=mesh,
        out_type=jax.ShapeDtypeStruct((B, D), table.dtype),
        scratch_types=[
            pltpu.VMEM((b_per_w,), jnp.int32),
            pltpu.VMEM((b_per_w, D), table.dtype),
            pltpu.SemaphoreType.DMA,
        ],
        compiler_params=pltpu.CompilerParams(skip_device_barrier=True),
    )
    def gather_kernel(table_hbm, idx_hbm, out_hbm, idx_v, rows_v, sem):
        wid = lax.axis_index("s") * nc + lax.axis_index("c")
        base = wid * b_per_w
        pltpu.sync_copy(idx_hbm.at[pl.ds(base, b_per_w)], idx_v)

        def issue(g, _):
            v16 = idx_v[pl.ds(g * 16, 16)]
            for l in range(16):
                pltpu.async_copy(table_hbm.at[v16[l]], rows_v.at[g * 16 + l], sem)
            return ()

        lax.fori_loop(0, b_per_w // 16, issue, ())
        pltpu.make_async_copy(table_hbm.at[pl.ds(0, b_per_w)], rows_v, sem).wait()
        pltpu.sync_copy(rows_v, out_hbm.at[pl.ds(base, b_per_w)])

    return gather_kernel(table, idx)


def _tc_loss(e, c, interpret=False):
    B, D = e.shape
    blk = 1024
    scale = _LN2 / (B * B)

    def body(c_ref, e_ref, out_ref):
        i = pl.program_id(0)
        cs = (c_ref[...] * -_LOG2E).astype(jnp.bfloat16)
        y = lax.dot_general(
            cs, e_ref[...].astype(jnp.bfloat16), (((1,), (1,)), ((), ())),
            preferred_element_type=jnp.float32,
        )
        part = jnp.sum(jnp.log2(1.0 + jnp.exp2(y))) * scale

        @pl.when(i == 0)
        def _():
            out_ref[0, 0] = 0.0

        out_ref[0, 0] += part

    out = pl.pallas_call(
        body,
        grid=(B // blk,),
        in_specs=[
            pl.BlockSpec((blk, D), lambda i: (i, 0)),
            pl.BlockSpec((B, D), lambda i: (0, 0)),
        ],
        out_specs=pl.BlockSpec(memory_space=pltpu.SMEM),
        out_shape=jax.ShapeDtypeStruct((1, 1), jnp.float32),
        interpret=interpret,
    )(c, e)
    return out[0, 0]


def kernel(inpt, trgs, emb_table, ffw_weight):
    inpt = inpt.astype(jnp.int32)
    trgs = trgs.astype(jnp.int32)
    emb_rm = _tc_transpose(emb_table.T)
    e = _sc_gather(emb_rm, inpt)
    ffw_rm = _tc_transpose(ffw_weight.T)
    c = _sc_gather(ffw_rm, trgs)
    return _tc_loss(e, c)


# speedup vs baseline: 1.0230x; 1.0230x over previous
import functools

import jax
import jax.numpy as jnp
from jax import lax
from jax.experimental import pallas as pl
from jax.experimental.pallas import tpu as pltpu
from jax.experimental.pallas import tpu_sc as plsc

_LOG2E = 1.4426950408889634
_LN2 = 0.6931471805599453


def _tc_transpose(x_t):
    D, V = x_t.shape
    blk = 16384

    def body(x_ref, o_ref):
        eye = jnp.eye(D, dtype=jnp.float32)
        o_ref[...] = lax.dot_general(
            x_ref[...], eye, (((0,), (0,)), ((), ())),
            preferred_element_type=jnp.float32,
        )

    return pl.pallas_call(
        body,
        grid=(pl.cdiv(V, blk),),
        in_specs=[pl.BlockSpec((D, blk), lambda i: (0, i))],
        out_specs=pl.BlockSpec((blk, D), lambda i: (i, 0)),
        out_shape=jax.ShapeDtypeStruct((V, D), jnp.float32),
    )(x_t)


def _sc_gather(table, idx):
    V, D = table.shape
    B = idx.shape[0]
    info = plsc.get_sparse_core_info()
    nc, ns = info.num_cores, info.num_subcores
    b_per_w = B // (nc * ns)
    mesh = plsc.VectorSubcoreMesh(core_axis_name="c", subcore_axis_name="s")

    @functools.partial(
        pl.kernel,
        mesh=mesh,
        out_type=jax.ShapeDtypeStruct((B, D), table.dtype),
        scratch_types=[
            pltpu.VMEM((b_per_w,), jnp.int32),
            pltpu.VMEM((b_per_w, D), table.dtype),
            pltpu.SemaphoreType.DMA,
        ],
        compiler_params=pltpu.CompilerParams(skip_device_barrier=True),
    )
    def gather_kernel(table_hbm, idx_hbm, out_hbm, idx_v, rows_v, sem):
        wid = lax.axis_index("s") * nc + lax.axis_index("c")
        base = wid * b_per_w
        pltpu.sync_copy(idx_hbm.at[pl.ds(base, b_per_w)], idx_v)

        def issue(g, _):
            v16 = idx_v[pl.ds(g * 16, 16)]
            for l in range(16):
                pltpu.async_copy(table_hbm.at[v16[l]], rows_v.at[g * 16 + l], sem)
            return ()

        lax.fori_loop(0, b_per_w // 16, issue, ())
        pltpu.make_async_copy(table_hbm.at[pl.ds(0, b_per_w)], rows_v, sem).wait()
        pltpu.sync_copy(rows_v, out_hbm.at[pl.ds(base, b_per_w)])

    return gather_kernel(table, idx)


def _tc_loss(e, c, interpret=False):
    B, D = e.shape
    blk = 2048
    scale = _LN2 / (B * B)

    def body(c_ref, e_ref, out_ref):
        i = pl.program_id(0)
        cs = (c_ref[...] * -_LOG2E).astype(jnp.bfloat16)
        y = lax.dot_general(
            cs, e_ref[...].astype(jnp.bfloat16), (((1,), (1,)), ((), ())),
            preferred_element_type=jnp.float32,
        )
        part = jnp.sum(jnp.log2(1.0 + jnp.exp2(y))) * scale

        @pl.when(i == 0)
        def _():
            out_ref[0, 0] = 0.0

        out_ref[0, 0] += part

    out = pl.pallas_call(
        body,
        grid=(B // blk,),
        in_specs=[
            pl.BlockSpec((blk, D), lambda i: (i, 0)),
            pl.BlockSpec((B, D), lambda i: (0, 0)),
        ],
        out_specs=pl.BlockSpec(memory_space=pltpu.SMEM),
        out_shape=jax.ShapeDtypeStruct((1, 1), jnp.float32),
        compiler_params=pltpu.CompilerParams(vmem_limit_bytes=100 * 1024 * 1024),
        interpret=interpret,
    )(c, e)
    return out[0, 0]


def kernel(inpt, trgs, emb_table, ffw_weight):
    inpt = inpt.astype(jnp.int32)
    trgs = trgs.astype(jnp.int32)
    emb_rm = _tc_transpose(emb_table.T)
    e = _sc_gather(emb_rm, inpt)
    ffw_rm = _tc_transpose(ffw_weight.T)
    c = _sc_gather(ffw_rm, trgs)
    return _tc_loss(e, c)


# speedup vs baseline: 1.0333x; 1.0101x over previous
import functools

import jax
import jax.numpy as jnp
from jax import lax
from jax.experimental import pallas as pl
from jax.experimental.pallas import tpu as pltpu
from jax.experimental.pallas import tpu_sc as plsc

_LOG2E = 1.4426950408889634
_LN2 = 0.6931471805599453


def _tc_transpose(x_t):
    D, V = x_t.shape
    blk = 32768

    def body(x_ref, o_ref):
        eye = jnp.eye(D, dtype=jnp.float32)
        o_ref[...] = lax.dot_general(
            x_ref[...], eye, (((0,), (0,)), ((), ())),
            preferred_element_type=jnp.float32,
        )

    return pl.pallas_call(
        body,
        grid=(pl.cdiv(V, blk),),
        in_specs=[pl.BlockSpec((D, blk), lambda i: (0, i))],
        out_specs=pl.BlockSpec((blk, D), lambda i: (i, 0)),
        out_shape=jax.ShapeDtypeStruct((V, D), jnp.float32),
        compiler_params=pltpu.CompilerParams(vmem_limit_bytes=100 * 1024 * 1024),
    )(x_t)


def _sc_gather(table, idx):
    V, D = table.shape
    B = idx.shape[0]
    info = plsc.get_sparse_core_info()
    nc, ns = info.num_cores, info.num_subcores
    b_per_w = B // (nc * ns)
    mesh = plsc.VectorSubcoreMesh(core_axis_name="c", subcore_axis_name="s")

    @functools.partial(
        pl.kernel,
        mesh=mesh,
        out_type=jax.ShapeDtypeStruct((B, D), table.dtype),
        scratch_types=[
            pltpu.VMEM((b_per_w,), jnp.int32),
            pltpu.VMEM((b_per_w, D), table.dtype),
            pltpu.SemaphoreType.DMA,
        ],
        compiler_params=pltpu.CompilerParams(skip_device_barrier=True),
    )
    def gather_kernel(table_hbm, idx_hbm, out_hbm, idx_v, rows_v, sem):
        wid = lax.axis_index("s") * nc + lax.axis_index("c")
        base = wid * b_per_w
        pltpu.sync_copy(idx_hbm.at[pl.ds(base, b_per_w)], idx_v)

        def issue(g, _):
            v16 = idx_v[pl.ds(g * 16, 16)]
            for l in range(16):
                pltpu.async_copy(table_hbm.at[v16[l]], rows_v.at[g * 16 + l], sem)
            return ()

        lax.fori_loop(0, b_per_w // 16, issue, ())
        pltpu.make_async_copy(table_hbm.at[pl.ds(0, b_per_w)], rows_v, sem).wait()
        pltpu.sync_copy(rows_v, out_hbm.at[pl.ds(base, b_per_w)])

    return gather_kernel(table, idx)


def _tc_loss(e, c, interpret=False):
    B, D = e.shape
    blk = 2048
    scale = _LN2 / (B * B)

    def body(c_ref, e_ref, out_ref):
        i = pl.program_id(0)
        cs = (c_ref[...] * -_LOG2E).astype(jnp.bfloat16)
        y = lax.dot_general(
            cs, e_ref[...].astype(jnp.bfloat16), (((1,), (1,)), ((), ())),
            preferred_element_type=jnp.float32,
        )
        part = jnp.sum(jnp.log2(1.0 + jnp.exp2(y))) * scale

        @pl.when(i == 0)
        def _():
            out_ref[0, 0] = 0.0

        out_ref[0, 0] += part

    out = pl.pallas_call(
        body,
        grid=(B // blk,),
        in_specs=[
            pl.BlockSpec((blk, D), lambda i: (i, 0)),
            pl.BlockSpec((B, D), lambda i: (0, 0)),
        ],
        out_specs=pl.BlockSpec(memory_space=pltpu.SMEM),
        out_shape=jax.ShapeDtypeStruct((1, 1), jnp.float32),
        compiler_params=pltpu.CompilerParams(vmem_limit_bytes=100 * 1024 * 1024),
        interpret=interpret,
    )(c, e)
    return out[0, 0]


def kernel(inpt, trgs, emb_table, ffw_weight):
    inpt = inpt.astype(jnp.int32)
    trgs = trgs.astype(jnp.int32)
    emb_rm = _tc_transpose(emb_table.T)
    e = _sc_gather(emb_rm, inpt)
    ffw_rm = _tc_transpose(ffw_weight.T)
    c = _sc_gather(ffw_rm, trgs)
    return _tc_loss(e, c)
